# baseline (speedup 1.0000x reference)
import functools

import numpy as np
import jax
import jax.numpy as jnp
from jax import lax
from jax.experimental import pallas as pl
from jax.experimental.pallas import tpu as pltpu
from jax.experimental.pallas import tpu_sc as plsc

N = 10000
E = 160000
IN_DIM = 8
HID = 16
OUT = 16
EDIM = 4

NC, NS = 2, 16
NW = NC * NS
EPW = E // NW
ROWS_PT = N // NS

EB = 3200
EBP = EB // 8

_EI3 = np.kron(np.eye(HID, dtype=np.float32), np.ones((1, 16), np.float32))
_S = np.kron(np.ones((HID, 1), np.float32), np.eye(OUT, dtype=np.float32))

_mesh = plsc.VectorSubcoreMesh(
    core_axis_name="c", subcore_axis_name="s", num_cores=NC, num_subcores=NS)
_SC_PARAMS = pltpu.CompilerParams(use_tc_tiling_on_sc=False)


@functools.partial(
    pl.kernel, mesh=_mesh, compiler_params=_SC_PARAMS,
    out_type=jax.ShapeDtypeStruct((E, HID), jnp.float32),
    scratch_types=[
        pltpu.VMEM((EPW,), jnp.int32),
        pltpu.VMEM((EPW, HID), jnp.float32),
        pltpu.SemaphoreType.DMA,
    ],
)
def _sc_gather(table_hbm, ei_hbm, out_hbm, idx_v, rows_v, sem):
    wid = lax.axis_index("s") * NC + lax.axis_index("c")
    base = wid * EPW
    pltpu.sync_copy(ei_hbm.at[0, pl.ds(base, EPW)], idx_v)
    pltpu.async_copy(table_hbm.at[idx_v], rows_v, sem).wait()
    pltpu.sync_copy(rows_v, out_hbm.at[pl.ds(base, EPW)])


@functools.partial(
    pl.kernel, mesh=_mesh, compiler_params=_SC_PARAMS,
    out_type=jax.ShapeDtypeStruct((NC, N, HID), jnp.float32),
    scratch_types=[
        pltpu.VMEM((EPW,), jnp.int32),
        pltpu.VMEM((EPW, HID), jnp.float32),
        pltpu.VMEM_SHARED((N, HID), jnp.float32),
    ],
)
def _sc_scatter(msg_hbm, ei_hbm, zeros_hbm, out_hbm, idx_v, vals_v, table_sh):
    cid = lax.axis_index("c")
    sid = lax.axis_index("s")
    wid = sid * NC + cid
    base = wid * EPW

    @pl.when(sid == 0)
    def _():
        pltpu.sync_copy(zeros_hbm, table_sh)

    plsc.subcore_barrier()
    pltpu.sync_copy(ei_hbm.at[1, pl.ds(base, EPW)], idx_v)
    pltpu.sync_copy(msg_hbm.at[pl.ds(base, EPW)], vals_v)
    pltpu.sync_copy(vals_v, table_sh.at[idx_v], add=True)
    plsc.subcore_barrier()
    rbase = sid * ROWS_PT
    pltpu.sync_copy(table_sh.at[pl.ds(rbase, ROWS_PT)],
                    out_hbm.at[cid, pl.ds(rbase, ROWS_PT)])


@functools.partial(
    pl.kernel, mesh=_mesh, compiler_params=_SC_PARAMS,
    out_type=jax.ShapeDtypeStruct((NC, N), jnp.float32),
    scratch_types=[
        pltpu.VMEM((EPW,), jnp.int32),
        pltpu.VMEM((EPW,), jnp.float32),
        pltpu.VMEM_SHARED((N,), jnp.float32),
    ],
)
def _sc_count(ei_hbm, zeros_hbm, ones_hbm, out_hbm, idx_v, ones_v, table_sh):
    cid = lax.axis_index("c")
    sid = lax.axis_index("s")
    wid = sid * NC + cid
    base = wid * EPW

    @pl.when(sid == 0)
    def _():
        pltpu.sync_copy(zeros_hbm, table_sh)

    plsc.subcore_barrier()
    pltpu.sync_copy(ei_hbm.at[1, pl.ds(base, EPW)], idx_v)
    pltpu.sync_copy(ones_hbm, ones_v)
    pltpu.sync_copy(ones_v, table_sh.at[idx_v], add=True)
    plsc.subcore_barrier()

    @pl.when(sid == 0)
    def _():
        pltpu.sync_copy(table_sh, out_hbm.at[cid])


def _tc_in_body(x_ref, w_ref, b_ref, out_ref):
    out_ref[...] = jnp.dot(x_ref[...], w_ref[...],
                           preferred_element_type=jnp.float32) + b_ref[...]


def _tc_in(x, W_in, b_in):
    return pl.pallas_call(
        _tc_in_body,
        out_shape=jax.ShapeDtypeStruct((N, HID), jnp.float32),
    )(x, W_in, b_in.reshape(1, HID))


def _tc_edge_body(eat_ref, xj_ref, w1_ref, b1_ref, w2_ref, brep_ref, ei_ref,
                  s_ref, out_ref):
    h2 = jnp.maximum(
        lax.dot_general(eat_ref[...], w1_ref[...], (((0,), (0,)), ((), ())),
                        preferred_element_type=jnp.float32)
        + b1_ref[...], 0.0)
    xj = xj_ref[...]
    g = (jnp.dot(h2, w2_ref[...], preferred_element_type=jnp.float32)
         + brep_ref[...])
    xrep = jnp.dot(xj, ei_ref[...], preferred_element_type=jnp.float32)
    out_ref[...] = jnp.dot(g * xrep, s_ref[...],
                           preferred_element_type=jnp.float32)


def _tc_edge(ea_t, xj, enW1, enb1, enW2, enb2):
    full = lambda i: (0, 0)
    return pl.pallas_call(
        _tc_edge_body,
        grid=(E // EB,),
        in_specs=[
            pl.BlockSpec((EDIM, EB), lambda i: (0, i)),
            pl.BlockSpec((EB, HID), lambda i: (i, 0)),
            pl.BlockSpec((EDIM, 32), full),
            pl.BlockSpec((1, 32), full),
            pl.BlockSpec((32, 256), full),
            pl.BlockSpec((1, 256), full),
            pl.BlockSpec((HID, 256), full),
            pl.BlockSpec((256, OUT), full),
        ],
        out_specs=pl.BlockSpec((EB, OUT), lambda i: (i, 0)),
        out_shape=jax.ShapeDtypeStruct((E, OUT), jnp.float32),
    )(ea_t, xj, enW1, enb1.reshape(1, 32), enW2, enb2.reshape(1, 256),
      _EI3, _S)


def _tc_node_body(part_ref, cnt_ref, h_ref, root_ref, bias_ref, w_ref, b_ref,
                  ms_ref, out_ref, *, leaky):
    s = part_ref[0] + part_ref[1]
    cnt = cnt_ref[0] + cnt_ref[1]
    inv = 1.0 / jnp.maximum(cnt, 1.0)
    y = (s * inv
         + jnp.dot(h_ref[...], root_ref[...], preferred_element_type=jnp.float32)
         + bias_ref[...])
    mean = jnp.mean(y, axis=0, keepdims=True)
    o = y - ms_ref[...] * mean
    var = jnp.mean(o * o, axis=0, keepdims=True)
    o = w_ref[...] * o * jax.lax.rsqrt(var + 1e-5) + b_ref[...]
    if leaky:
        o = jnp.where(o >= 0, o, 0.2 * o)
    out_ref[...] = o


def _tc_node(part, cnt_part, h_prev, root, bias, gn_w, gn_b, gn_ms, leaky):
    d = root.shape[1]
    return pl.pallas_call(
        functools.partial(_tc_node_body, leaky=leaky),
        out_shape=jax.ShapeDtypeStruct((N, d), jnp.float32),
    )(part, cnt_part.reshape(NC, N, 1), h_prev, root, bias.reshape(1, d),
      gn_w.reshape(1, d), gn_b.reshape(1, d), gn_ms.reshape(1, d))


def kernel(x, edge_index, edge_attr, W_in, b_in, en1_W1, en1_b1, en1_W2, en1_b2,
           root1, bias1, gn1_w, gn1_b, gn1_ms, en2_W1, en2_b1, en2_W2, en2_b2,
           root2, bias2, gn2_w, gn2_b, gn2_ms):
    zeros2d = jnp.zeros((N, HID), jnp.float32)
    zeros1d = jnp.zeros((N,), jnp.float32)
    ones1d = jnp.ones((EPW,), jnp.float32)

    h = _tc_in(x, W_in, b_in)
    cnt_part = _sc_count(edge_index, zeros1d, ones1d)
    ea_t = edge_attr.T

    xj1 = _sc_gather(h, edge_index)
    msg1 = _tc_edge(ea_t, xj1, en1_W1, en1_b1, en1_W2, en1_b2)
    part1 = _sc_scatter(msg1, edge_index, zeros2d)
    h1 = _tc_node(part1, cnt_part, h, root1, bias1, gn1_w, gn1_b, gn1_ms,
                  leaky=True)

    xj2 = _sc_gather(h1, edge_index)
    msg2 = _tc_edge(ea_t, xj2, en2_W1, en2_b1, en2_W2, en2_b2)
    part2 = _sc_scatter(msg2, edge_index, zeros2d)
    return _tc_node(part2, cnt_part, h1, root2, bias2, gn2_w, gn2_b, gn2_ms,
                    leaky=False)


# speedup vs baseline: 1.4585x; 1.4585x over previous
import functools

import numpy as np
import jax
import jax.numpy as jnp
from jax import lax
from jax.experimental import pallas as pl
from jax.experimental.pallas import tpu as pltpu
from jax.experimental.pallas import tpu_sc as plsc

N = 10000
E = 160000
IN_DIM = 8
HID = 16
OUT = 16
EDIM = 4

NC, NS = 2, 16
NW = NC * NS
EPW = E // NW
ROWS_PT = N // NS

EB = 3200
EBP = EB // 8

_EI3 = np.kron(np.eye(HID, dtype=np.float32), np.ones((1, 16), np.float32))
_S = np.kron(np.ones((HID, 1), np.float32), np.eye(OUT, dtype=np.float32))

_mesh = plsc.VectorSubcoreMesh(
    core_axis_name="c", subcore_axis_name="s", num_cores=NC, num_subcores=NS)
_SC_PARAMS = pltpu.CompilerParams(use_tc_tiling_on_sc=False)


@functools.partial(
    pl.kernel, mesh=_mesh, compiler_params=_SC_PARAMS,
    out_type=jax.ShapeDtypeStruct((E, HID), jnp.float32),
    scratch_types=[
        pltpu.VMEM((EPW,), jnp.int32),
        pltpu.VMEM((EPW, HID), jnp.float32),
        pltpu.SemaphoreType.DMA,
    ],
)
def _sc_gather(table_hbm, ei_hbm, out_hbm, idx_v, rows_v, sem):
    wid = lax.axis_index("s") * NC + lax.axis_index("c")
    base = wid * EPW
    pltpu.sync_copy(ei_hbm.at[0, pl.ds(base, EPW)], idx_v)
    pltpu.async_copy(table_hbm.at[idx_v], rows_v, sem).wait()
    pltpu.sync_copy(rows_v, out_hbm.at[pl.ds(base, EPW)])


@functools.partial(
    pl.kernel, mesh=_mesh, compiler_params=_SC_PARAMS,
    out_type=jax.ShapeDtypeStruct((NC, N, HID), jnp.float32),
    scratch_types=[
        pltpu.VMEM((EPW,), jnp.int32),
        pltpu.VMEM((EPW, HID), jnp.float32),
        pltpu.VMEM_SHARED((N, HID), jnp.float32),
    ],
)
def _sc_scatter(msg_hbm, ei_hbm, zeros_hbm, out_hbm, idx_v, vals_v, table_sh):
    cid = lax.axis_index("c")
    sid = lax.axis_index("s")
    wid = sid * NC + cid
    base = wid * EPW

    @pl.when(sid == 0)
    def _():
        pltpu.sync_copy(zeros_hbm, table_sh)

    plsc.subcore_barrier()
    pltpu.sync_copy(ei_hbm.at[1, pl.ds(base, EPW)], idx_v)
    pltpu.sync_copy(msg_hbm.at[pl.ds(base, EPW)], vals_v)
    pltpu.sync_copy(vals_v, table_sh.at[idx_v], add=True)
    plsc.subcore_barrier()
    rbase = sid * ROWS_PT
    pltpu.sync_copy(table_sh.at[pl.ds(rbase, ROWS_PT)],
                    out_hbm.at[cid, pl.ds(rbase, ROWS_PT)])


@functools.partial(
    pl.kernel, mesh=_mesh, compiler_params=_SC_PARAMS,
    out_type=jax.ShapeDtypeStruct((NC, N), jnp.float32),
    scratch_types=[
        pltpu.VMEM((EPW,), jnp.int32),
        pltpu.VMEM((EPW,), jnp.float32),
        pltpu.VMEM_SHARED((N,), jnp.float32),
    ],
)
def _sc_count(ei_hbm, zeros_hbm, ones_hbm, out_hbm, idx_v, ones_v, table_sh):
    cid = lax.axis_index("c")
    sid = lax.axis_index("s")
    wid = sid * NC + cid
    base = wid * EPW

    @pl.when(sid == 0)
    def _():
        pltpu.sync_copy(zeros_hbm, table_sh)

    plsc.subcore_barrier()
    pltpu.sync_copy(ei_hbm.at[1, pl.ds(base, EPW)], idx_v)
    pltpu.sync_copy(ones_hbm, ones_v)
    pltpu.sync_copy(ones_v, table_sh.at[idx_v], add=True)
    plsc.subcore_barrier()

    @pl.when(sid == 0)
    def _():
        pltpu.sync_copy(table_sh, out_hbm.at[cid])


def _tc_in_body(x_ref, w_ref, b_ref, out_ref):
    out_ref[...] = jnp.dot(x_ref[...], w_ref[...],
                           preferred_element_type=jnp.float32) + b_ref[...]


def _tc_in(x, W_in, b_in):
    return pl.pallas_call(
        _tc_in_body,
        out_shape=jax.ShapeDtypeStruct((N, HID), jnp.float32),
    )(x, W_in, b_in.reshape(1, HID))


def _tc_edge_body(eat_ref, xjp_ref, w1_ref, b1_ref, w2_ref, brep_ref, ei_ref,
                  s_ref, out_ref):
    h2r = jnp.maximum(
        lax.dot_general(eat_ref[...], w1_ref[...], (((0,), (0,)), ((), ())),
                        preferred_element_type=jnp.float32)
        + b1_ref[...], 0.0)
    h3 = jnp.reshape(h2r, (EBP, 8, 32))
    h2 = jnp.concatenate([h3[:, j, :] for j in range(8)], axis=0)
    xjp = xjp_ref[...]
    xj = jnp.concatenate([xjp[:, 16 * j:16 * (j + 1)] for j in range(8)],
                         axis=0)
    g = (jnp.dot(h2, w2_ref[...], preferred_element_type=jnp.float32)
         + brep_ref[...])
    xrep = jnp.dot(xj, ei_ref[...], preferred_element_type=jnp.float32)
    msg = jnp.dot(g * xrep, s_ref[...], preferred_element_type=jnp.float32)
    out_ref[...] = jnp.concatenate(
        [msg[j * EBP:(j + 1) * EBP, :] for j in range(8)], axis=1)


def _tc_edge(ea_t, xjp, enW1, enb1, enW2, enb2):
    full = lambda i: (0, 0)
    return pl.pallas_call(
        _tc_edge_body,
        grid=(E // EB,),
        in_specs=[
            pl.BlockSpec((EDIM, EB), lambda i: (0, i)),
            pl.BlockSpec((EBP, 128), lambda i: (i, 0)),
            pl.BlockSpec((EDIM, 32), full),
            pl.BlockSpec((1, 32), full),
            pl.BlockSpec((32, 256), full),
            pl.BlockSpec((1, 256), full),
            pl.BlockSpec((HID, 256), full),
            pl.BlockSpec((256, OUT), full),
        ],
        out_specs=pl.BlockSpec((EBP, 128), lambda i: (i, 0)),
        out_shape=jax.ShapeDtypeStruct((E // 8, 128), jnp.float32),
    )(ea_t, xjp, enW1, enb1.reshape(1, 32), enW2, enb2.reshape(1, 256),
      _EI3, _S)


def _tc_node_body(part_ref, cnt_ref, h_ref, root_ref, bias_ref, w_ref, b_ref,
                  ms_ref, out_ref, *, leaky):
    s = part_ref[0] + part_ref[1]
    cnt = cnt_ref[0] + cnt_ref[1]
    inv = 1.0 / jnp.maximum(cnt, 1.0)
    y = (s * inv
         + jnp.dot(h_ref[...], root_ref[...], preferred_element_type=jnp.float32)
         + bias_ref[...])
    mean = jnp.mean(y, axis=0, keepdims=True)
    o = y - ms_ref[...] * mean
    var = jnp.mean(o * o, axis=0, keepdims=True)
    o = w_ref[...] * o * jax.lax.rsqrt(var + 1e-5) + b_ref[...]
    if leaky:
        o = jnp.where(o >= 0, o, 0.2 * o)
    out_ref[...] = o


def _tc_node(part, cnt_part, h_prev, root, bias, gn_w, gn_b, gn_ms, leaky):
    d = root.shape[1]
    return pl.pallas_call(
        functools.partial(_tc_node_body, leaky=leaky),
        out_shape=jax.ShapeDtypeStruct((N, d), jnp.float32),
    )(part, cnt_part.reshape(NC, N, 1), h_prev, root, bias.reshape(1, d),
      gn_w.reshape(1, d), gn_b.reshape(1, d), gn_ms.reshape(1, d))


def kernel(x, edge_index, edge_attr, W_in, b_in, en1_W1, en1_b1, en1_W2, en1_b2,
           root1, bias1, gn1_w, gn1_b, gn1_ms, en2_W1, en2_b1, en2_W2, en2_b2,
           root2, bias2, gn2_w, gn2_b, gn2_ms):
    zeros2d = jnp.zeros((N, HID), jnp.float32)
    zeros1d = jnp.zeros((N,), jnp.float32)
    ones1d = jnp.ones((EPW,), jnp.float32)

    h = _tc_in(x, W_in, b_in)
    cnt_part = _sc_count(edge_index, zeros1d, ones1d)
    ea_t = edge_attr.T

    xj1 = _sc_gather(h, edge_index).reshape(E // 8, 128)
    msg1 = _tc_edge(ea_t, xj1, en1_W1, en1_b1, en1_W2, en1_b2)
    part1 = _sc_scatter(msg1.reshape(E, HID), edge_index, zeros2d)
    h1 = _tc_node(part1, cnt_part, h, root1, bias1, gn1_w, gn1_b, gn1_ms,
                  leaky=True)

    xj2 = _sc_gather(h1, edge_index).reshape(E // 8, 128)
    msg2 = _tc_edge(ea_t, xj2, en2_W1, en2_b1, en2_W2, en2_b2)
    part2 = _sc_scatter(msg2.reshape(E, HID), edge_index, zeros2d)
    return _tc_node(part2, cnt_part, h1, root2, bias2, gn2_w, gn2_b, gn2_ms,
                    leaky=False)
